# baseline (speedup 1.0000x reference)
import functools

import jax
import jax.numpy as jnp
import numpy as np
from jax import lax
from jax.experimental import pallas as pl
from jax.experimental.pallas import tpu as pltpu
from jax.experimental.pallas import tpu_sc as plsc

B, N, H, K = 4, 8192, 768, 3
BN = B * N
HW = H // 2
TM = 1024
NW = 32
RW = N // NW
T = 16
NB = RW // T


def _tc_body(x_ref, w2_ref, womt_ref, y_ref, idx_ref, coef_ref):
    pid = pl.program_id(0)
    k = pl.program_id(1)
    x = x_ref[...]
    yf = jnp.dot(
        x.astype(jnp.bfloat16), w2_ref[k], preferred_element_type=jnp.float32
    )
    a_bits = jax.lax.bitcast_convert_type(yf[:, :HW], jnp.int32)
    b_bits = jax.lax.bitcast_convert_type(yf[:, HW:], jnp.int32)
    half = jnp.int32(0x8000)
    lo = ((a_bits + half) >> 16) & jnp.int32(0xFFFF)
    hi = (b_bits + half) & jnp.int32(-65536)
    y_ref[...] = hi | lo

    @pl.when(k == 0)
    def _emit_indices():
        om = jnp.dot(x, womt_ref[...], preferred_element_type=jnp.float32)
        off = om[:, :K]
        m = jax.nn.sigmoid(om[:, K:])
        i0 = pid * TM
        ii = jax.lax.broadcasted_iota(jnp.int32, (TM, K), 0) + i0
        ki = jax.lax.broadcasted_iota(jnp.int32, (TM, K), 1)
        pos = (ii + ki).astype(jnp.float32) - float(K // 2) + off
        valid = (pos >= 0.0) & (pos < float(N))
        ff = jnp.floor(pos)
        fi = jnp.clip(ff.astype(jnp.int32), 0, N - 1)
        ci = jnp.minimum(fi + 1, N - 1)
        wc = pos - ff
        mv = m * valid.astype(jnp.float32)
        kn = ki * N
        idx_ref[...] = jnp.concatenate([kn + fi, kn + ci], axis=1)
        coef_ref[...] = jnp.concatenate([mv * (1.0 - wc), mv * wc], axis=1)


def _tc_call(x2, w2p, womt, b):
    nb = N // TM
    return pl.pallas_call(
        _tc_body,
        grid=(nb, K),
        in_specs=[
            pl.BlockSpec((TM, H), lambda p, k, o=b * nb: (p + o, 0)),
            pl.BlockSpec((K, H, H), lambda p, k: (0, 0, 0)),
            pl.BlockSpec((H, 2 * K), lambda p, k: (0, 0)),
        ],
        out_specs=[
            pl.BlockSpec((TM, HW), lambda p, k: (k * nb + p, 0)),
            pl.BlockSpec((TM, 2 * K), lambda p, k: (p, 0)),
            pl.BlockSpec((TM, 2 * K), lambda p, k: (p, 0)),
        ],
        out_shape=[
            jax.ShapeDtypeStruct((K * N, HW), jnp.int32),
            jax.ShapeDtypeStruct((N, 2 * K), jnp.int32),
            jax.ShapeDtypeStruct((N, 2 * K), jnp.float32),
        ],
    )(x2, w2p, womt)


@functools.cache
def _sc_gather_kernel():
    mesh = plsc.VectorSubcoreMesh(core_axis_name="c", subcore_axis_name="s")
    return pl.kernel(
        _sc_gather_body,
        mesh=mesh,
        compiler_params=pltpu.CompilerParams(needs_layout_passes=False),
        out_type=jax.ShapeDtypeStruct((N, H), jnp.float32),
        scratch_types=[
            pltpu.VMEM((NB, T * 6), jnp.int32),
            pltpu.VMEM((NB, T * 6), jnp.float32),
            pltpu.VMEM((T * 6, HW), jnp.int32),
            pltpu.VMEM((T * 6, HW), jnp.int32),
            pltpu.VMEM((T, H), jnp.float32),
            pltpu.VMEM((T, H), jnp.float32),
            pltpu.SemaphoreType.DMA,
            pltpu.SemaphoreType.DMA,
            pltpu.SemaphoreType.DMA,
            pltpu.SemaphoreType.DMA,
        ],
    )


def _sc_gather_body(y_hbm, idx_hbm, coef_hbm, out_hbm,
                    idx_v, coef_v, rows0, rows1, outv0, outv1,
                    gsem0, gsem1, osem0, osem1):
    wid = lax.axis_index("s") * 2 + lax.axis_index("c")
    rows = (rows0, rows1)
    outv = (outv0, outv1)
    gsem = (gsem0, gsem1)
    osem = (osem0, osem1)

    pltpu.sync_copy(idx_hbm.at[wid], idx_v)
    pltpu.sync_copy(coef_hbm.at[wid], coef_v)

    def gather_desc(i, p):
        return pltpu.make_async_copy(y_hbm.at[idx_v.at[i]], rows[p], gsem[p])

    def out_desc(i, p):
        base = wid * RW + i * T
        return pltpu.make_async_copy(outv[p], out_hbm.at[pl.ds(base, T)], osem[p])

    gather_desc(0, 0).start()
    gather_desc(1, 1).start()

    def outer(j, carry):
        for p in range(2):
            i = j * 2 + p
            gather_desc(i, p).wait()

            @pl.when(i >= 2)
            def _wait_out(i=i, p=p):
                out_desc(i - 2, p).wait()

            cvecs = [coef_v[i, pl.ds(16 * q, 16)] for q in range(T * 6 // 16)]
            for t in range(T):
                cs = [cvecs[(6 * t + j2) // 16][(6 * t + j2) % 16] for j2 in range(6)]

                @plsc.parallel_loop(0, H // 32, unroll=4)
                def chunk(c, t=t, p=p, cs=cs):
                    acc_a = jnp.zeros((16,), jnp.float32)
                    acc_b = jnp.zeros((16,), jnp.float32)
                    for j2 in range(6):
                        v = rows[p][6 * t + j2, pl.ds(c * 16, 16)]
                        a = plsc.bitcast(v << 16, jnp.float32)
                        b = plsc.bitcast(v & jnp.int32(-65536), jnp.float32)
                        acc_a = acc_a + cs[j2] * a
                        acc_b = acc_b + cs[j2] * b
                    outv[p][t, pl.ds(c * 32, 16)] = acc_a
                    outv[p][t, pl.ds(c * 32 + 16, 16)] = acc_b

            @pl.when(i + 2 < NB)
            def _next_gather(i=i, p=p):
                gather_desc(i + 2, p).start()

            out_desc(i, p).start()
        return carry

    lax.fori_loop(0, NB // 2, outer, 0)
    out_desc(NB - 2, 0).wait()
    out_desc(NB - 1, 1).wait()


_PERM2 = np.arange(H).reshape(H // 32, 2, 16)
_COL_PERM = np.concatenate([_PERM2[:, 0, :].ravel(), _PERM2[:, 1, :].ravel()])


def kernel(basic_units, W_off, W_mod, weight):
    x2 = basic_units.reshape(BN, H)
    w2p = jnp.transpose(weight.astype(jnp.bfloat16), (2, 1, 0))[:, :, _COL_PERM]
    womt = jnp.concatenate([W_off, W_mod], axis=0).T
    sc = _sc_gather_kernel()
    outs = []
    for b in range(B):
        y, idx6, coef6 = _tc_call(x2, w2p, womt, b)
        outs.append(sc(
            y,
            idx6.reshape(NW, NB, T * 6),
            coef6.reshape(NW, NB, T * 6),
        ))
    return jnp.stack(outs)


# speedup vs baseline: 1.0310x; 1.0310x over previous
import functools

import jax
import jax.numpy as jnp
import numpy as np
from jax import lax
from jax.experimental import pallas as pl
from jax.experimental.pallas import tpu as pltpu
from jax.experimental.pallas import tpu_sc as plsc

B, N, H, K = 4, 8192, 768, 3
BN = B * N
HW = H // 2
TM = 1024
NW = 32
RW = N // NW
T = 8
NB = RW // T


def _tc_body(x_ref, w2_ref, womt_ref, y_ref, idx_ref, coef_ref):
    pid = pl.program_id(0)
    k = pl.program_id(1)
    x = x_ref[...]
    yf = jnp.dot(
        x.astype(jnp.bfloat16), w2_ref[k], preferred_element_type=jnp.float32
    )
    a_bits = jax.lax.bitcast_convert_type(yf[:, :HW], jnp.int32)
    b_bits = jax.lax.bitcast_convert_type(yf[:, HW:], jnp.int32)
    half = jnp.int32(0x8000)
    lo = ((a_bits + half) >> 16) & jnp.int32(0xFFFF)
    hi = (b_bits + half) & jnp.int32(-65536)
    y_ref[...] = hi | lo

    @pl.when(k == 0)
    def _emit_indices():
        om = jnp.dot(x, womt_ref[...], preferred_element_type=jnp.float32)
        off = om[:, :K]
        m = jax.nn.sigmoid(om[:, K:])
        i0 = pid * TM
        ii = jax.lax.broadcasted_iota(jnp.int32, (TM, K), 0) + i0
        ki = jax.lax.broadcasted_iota(jnp.int32, (TM, K), 1)
        pos = (ii + ki).astype(jnp.float32) - float(K // 2) + off
        valid = (pos >= 0.0) & (pos < float(N))
        ff = jnp.floor(pos)
        fi = jnp.clip(ff.astype(jnp.int32), 0, N - 1)
        ci = jnp.minimum(fi + 1, N - 1)
        wc = pos - ff
        mv = m * valid.astype(jnp.float32)
        kn = ki * N
        idx_ref[...] = jnp.concatenate([kn + fi, kn + ci], axis=1)
        coef_ref[...] = jnp.concatenate([mv * (1.0 - wc), mv * wc], axis=1)


def _tc_call(x2, w2p, womt, b):
    nb = N // TM
    return pl.pallas_call(
        _tc_body,
        grid=(nb, K),
        in_specs=[
            pl.BlockSpec((TM, H), lambda p, k, o=b * nb: (p + o, 0)),
            pl.BlockSpec((K, H, H), lambda p, k: (0, 0, 0)),
            pl.BlockSpec((H, 2 * K), lambda p, k: (0, 0)),
        ],
        out_specs=[
            pl.BlockSpec((TM, HW), lambda p, k: (k * nb + p, 0)),
            pl.BlockSpec((TM, 2 * K), lambda p, k: (p, 0)),
            pl.BlockSpec((TM, 2 * K), lambda p, k: (p, 0)),
        ],
        out_shape=[
            jax.ShapeDtypeStruct((K * N, HW), jnp.int32),
            jax.ShapeDtypeStruct((N, 2 * K), jnp.int32),
            jax.ShapeDtypeStruct((N, 2 * K), jnp.float32),
        ],
    )(x2, w2p, womt)


@functools.cache
def _sc_gather_kernel():
    mesh = plsc.VectorSubcoreMesh(core_axis_name="c", subcore_axis_name="s")
    return pl.kernel(
        _sc_gather_body,
        mesh=mesh,
        compiler_params=pltpu.CompilerParams(needs_layout_passes=False),
        out_type=jax.ShapeDtypeStruct((N, H), jnp.float32),
        scratch_types=[
            pltpu.VMEM((NB, T * 6), jnp.int32),
            pltpu.VMEM((NB, T * 6), jnp.float32),
            pltpu.VMEM((T * 6, HW), jnp.int32),
            pltpu.VMEM((T * 6, HW), jnp.int32),
            pltpu.VMEM((T * 6, HW), jnp.int32),
            pltpu.VMEM((T * 6, HW), jnp.int32),
            pltpu.VMEM((T, H), jnp.float32),
            pltpu.VMEM((T, H), jnp.float32),
            pltpu.VMEM((T, H), jnp.float32),
            pltpu.VMEM((T, H), jnp.float32),
            pltpu.SemaphoreType.DMA,
            pltpu.SemaphoreType.DMA,
            pltpu.SemaphoreType.DMA,
            pltpu.SemaphoreType.DMA,
            pltpu.SemaphoreType.DMA,
            pltpu.SemaphoreType.DMA,
            pltpu.SemaphoreType.DMA,
            pltpu.SemaphoreType.DMA,
        ],
    )


_RING = 4


def _sc_gather_body(y_hbm, idx_hbm, coef_hbm, out_hbm,
                    idx_v, coef_v, rows0, rows1, rows2, rows3,
                    outv0, outv1, outv2, outv3,
                    gsem0, gsem1, gsem2, gsem3,
                    osem0, osem1, osem2, osem3):
    wid = lax.axis_index("s") * 2 + lax.axis_index("c")
    rows = (rows0, rows1, rows2, rows3)
    outv = (outv0, outv1, outv2, outv3)
    gsem = (gsem0, gsem1, gsem2, gsem3)
    osem = (osem0, osem1, osem2, osem3)

    pltpu.sync_copy(idx_hbm.at[wid], idx_v)
    pltpu.sync_copy(coef_hbm.at[wid], coef_v)

    def gather_desc(i, p):
        return pltpu.make_async_copy(y_hbm.at[idx_v.at[i]], rows[p], gsem[p])

    def out_desc(i, p):
        base = wid * RW + i * T
        return pltpu.make_async_copy(outv[p], out_hbm.at[pl.ds(base, T)], osem[p])

    for p in range(_RING):
        gather_desc(p, p).start()

    def outer(j, carry):
        for p in range(_RING):
            i = j * _RING + p
            gather_desc(i, p).wait()

            @pl.when(i >= _RING)
            def _wait_out(i=i, p=p):
                out_desc(i - _RING, p).wait()

            cvecs = [coef_v[i, pl.ds(16 * q, 16)] for q in range(T * 6 // 16)]
            for t in range(T):
                cs = [cvecs[(6 * t + j2) // 16][(6 * t + j2) % 16] for j2 in range(6)]

                @plsc.parallel_loop(0, H // 32, unroll=4)
                def chunk(c, t=t, p=p, cs=cs):
                    acc_a = jnp.zeros((16,), jnp.float32)
                    acc_b = jnp.zeros((16,), jnp.float32)
                    for j2 in range(6):
                        v = rows[p][6 * t + j2, pl.ds(c * 16, 16)]
                        a = plsc.bitcast(v << 16, jnp.float32)
                        b = plsc.bitcast(v & jnp.int32(-65536), jnp.float32)
                        acc_a = acc_a + cs[j2] * a
                        acc_b = acc_b + cs[j2] * b
                    outv[p][t, pl.ds(c * 32, 16)] = acc_a
                    outv[p][t, pl.ds(c * 32 + 16, 16)] = acc_b

            @pl.when(i + _RING < NB)
            def _next_gather(i=i, p=p):
                gather_desc(i + _RING, p).start()

            out_desc(i, p).start()
        return carry

    lax.fori_loop(0, NB // _RING, outer, 0)
    for p in range(_RING):
        out_desc(NB - _RING + p, p).wait()


_PERM2 = np.arange(H).reshape(H // 32, 2, 16)
_COL_PERM = np.concatenate([_PERM2[:, 0, :].ravel(), _PERM2[:, 1, :].ravel()])


def kernel(basic_units, W_off, W_mod, weight):
    x2 = basic_units.reshape(BN, H)
    w2p = jnp.transpose(weight.astype(jnp.bfloat16), (2, 1, 0))[:, :, _COL_PERM]
    womt = jnp.concatenate([W_off, W_mod], axis=0).T
    sc = _sc_gather_kernel()
    outs = []
    for b in range(B):
        y, idx6, coef6 = _tc_call(x2, w2p, womt, b)
        outs.append(sc(
            y,
            idx6.reshape(NW, NB, T * 6),
            coef6.reshape(NW, NB, T * 6),
        ))
    return jnp.stack(outs)
